# Initial kernel scaffold; baseline (speedup 1.0000x reference)
import jax
import jax.numpy as jnp
from jax.experimental import pallas as pl


def kernel(objectness, bbox_deltas, anchors):
    raise NotImplementedError("write your pallas kernel here")


# speedup vs baseline: 23.5735x; 23.5735x over previous
import jax
import jax.numpy as jnp
import math
from jax.experimental import pallas as pl

_IMG = 800.0
_PRE_N = 1000
_NMS_TH = 0.7
_MIN_SIZE = 1e-3
_BBOX_CLIP = math.log(1000.0 / 16.0)
_PAD = 1024


def _rpn_nms_kernel(d_ref, a_ref, s_ref, o_ref):
    o_ref[0] = jnp.zeros((_PAD, 128), jnp.float32)

    dx = d_ref[0, 0]
    dy = d_ref[0, 1]
    dw = jnp.minimum(d_ref[0, 2], _BBOX_CLIP)
    dh = jnp.minimum(d_ref[0, 3], _BBOX_CLIP)
    ax1 = a_ref[0, 0]
    ay1 = a_ref[0, 1]
    ax2 = a_ref[0, 2]
    ay2 = a_ref[0, 3]

    aw = ax2 - ax1
    ah = ay2 - ay1
    acx = ax1 + 0.5 * aw
    acy = ay1 + 0.5 * ah
    pcx = dx * aw + acx
    pcy = dy * ah + acy
    pw = jnp.exp(dw) * aw
    ph = jnp.exp(dh) * ah

    x1 = jnp.clip(pcx - 0.5 * pw, 0.0, _IMG)
    y1 = jnp.clip(pcy - 0.5 * ph, 0.0, _IMG)
    x2 = jnp.clip(pcx + 0.5 * pw, 0.0, _IMG)
    y2 = jnp.clip(pcy + 0.5 * ph, 0.0, _IMG)
    prob = jax.nn.sigmoid(s_ref[0])

    rowi = jax.lax.broadcasted_iota(jnp.int32, (8, 128), 0)
    coli = jax.lax.broadcasted_iota(jnp.int32, (8, 128), 1)
    flat = rowi * 128 + coli
    valid = (flat < _PRE_N) & ((x2 - x1) >= _MIN_SIZE) & ((y2 - y1) >= _MIN_SIZE)
    area = (x2 - x1) * (y2 - y1)
    lane = jax.lax.broadcasted_iota(jnp.int32, (1, 128), 1)

    def step(j, carry):
        keep, k = carry
        sel = flat == j

        def ext(v):
            return jnp.max(jnp.where(sel, v, -1e30))

        alive = jnp.max(jnp.where(sel, keep, 0.0)) > 0.5
        bx1 = ext(x1)
        by1 = ext(y1)
        bx2 = ext(x2)
        by2 = ext(y2)
        pj = ext(prob)
        aj = (bx2 - bx1) * (by2 - by1)

        xx1 = jnp.maximum(bx1, x1)
        yy1 = jnp.maximum(by1, y1)
        xx2 = jnp.minimum(bx2, x2)
        yy2 = jnp.minimum(by2, y2)
        inter = jnp.maximum(xx2 - xx1, 0.0) * jnp.maximum(yy2 - yy1, 0.0)
        iou = inter / (aj + area - inter + 1e-9)
        suppress = alive & (iou > _NMS_TH) & (flat > j)
        keep = jnp.where(suppress, 0.0, keep)

        row = jnp.where(
            lane == 0,
            bx1,
            jnp.where(
                lane == 1,
                by1,
                jnp.where(
                    lane == 2,
                    bx2,
                    jnp.where(lane == 3, by2, jnp.where(lane == 4, pj, 0.0)),
                ),
            ),
        )
        slot = jnp.where(alive, k, _PAD - 1)
        o_ref[0, pl.ds(slot, 1), :] = row
        k = k + jnp.where(alive, 1, 0)
        return keep, k

    jax.lax.fori_loop(
        0, _PRE_N, step, (valid.astype(jnp.float32), jnp.int32(0))
    )


def kernel(objectness, bbox_deltas, anchors):
    n, a, h, w = objectness.shape
    hwa = h * w * a
    obj = objectness.transpose(0, 2, 3, 1).reshape(n, hwa)
    deltas = (
        bbox_deltas.reshape(n, a, 4, h, w)
        .transpose(0, 3, 4, 1, 2)
        .reshape(n, hwa, 4)
    )
    scores, idx = jax.lax.top_k(obj, _PRE_N)
    sel_d = jnp.take_along_axis(deltas, idx[:, :, None], axis=1)
    sel_a = jnp.take(anchors, idx, axis=0)

    pad = _PAD - _PRE_N
    d_in = (
        jnp.pad(sel_d, ((0, 0), (0, pad), (0, 0)))
        .transpose(0, 2, 1)
        .reshape(n, 4, 8, 128)
    )
    a_in = (
        jnp.pad(sel_a, ((0, 0), (0, pad), (0, 0)))
        .transpose(0, 2, 1)
        .reshape(n, 4, 8, 128)
    )
    s_in = jnp.pad(scores, ((0, 0), (0, pad))).reshape(n, 8, 128)

    out = pl.pallas_call(
        _rpn_nms_kernel,
        grid=(n,),
        in_specs=[
            pl.BlockSpec((1, 4, 8, 128), lambda i: (i, 0, 0, 0)),
            pl.BlockSpec((1, 4, 8, 128), lambda i: (i, 0, 0, 0)),
            pl.BlockSpec((1, 8, 128), lambda i: (i, 0, 0)),
        ],
        out_specs=pl.BlockSpec((1, _PAD, 128), lambda i: (i, 0, 0)),
        out_shape=jax.ShapeDtypeStruct((n, _PAD, 128), jnp.float32),
    )(d_in, a_in, s_in)
    return out[:, :_PRE_N, :5]
